# baseline (speedup 1.0000x reference)
import jax
import jax.numpy as jnp
from jax.experimental import pallas as pl
from jax.experimental.pallas import tpu as pltpu

_DIM = 1024
_H = 16
_DH = 64
_W = 64
_CBS = 32
_SBS = 32
_NSEL = 16
_S = 2048
_NBC = _S // _CBS
_NQ = _S // _SBS
_CH = 256
_NC = _S // _CH
_SCALE = _DH ** -0.5
_NEG = -1e9

_HIGH = jax.lax.Precision.HIGHEST
_DEF = jax.lax.Precision.DEFAULT


def _dotT(a, b, precision=_DEF):
    return jax.lax.dot_general(a, b, (((1,), (1,)), ((), ())),
                               precision=precision,
                               preferred_element_type=jnp.float32)


def _dot(a, b, precision=_DEF):
    return jax.lax.dot_general(a, b, (((1,), (0,)), ((), ())),
                               precision=precision,
                               preferred_element_type=jnp.float32)


def _b16(a):
    return a.astype(jnp.bfloat16)


def _softmax_parts(x):
    e = jnp.exp(x)
    return e, 1.0 / jnp.sum(e, axis=1, keepdims=True)


def _proj_group(grp, x_ref, wq_ref, wk_ref, wv_ref, wg_ref,
                q_scr, kb_scr, vb_scr, kc_scr, vc_scr, g_scr):
    _RC = 512
    nb = _RC // _CBS
    pr = jax.lax.broadcasted_iota(jnp.int32, (nb, _RC), 0)
    pc = jax.lax.broadcasted_iota(jnp.int32, (nb, _RC), 1)
    Pc = jnp.where(pc // _CBS == pr, 1.0 / _CBS, 0.0)

    wq = wq_ref[...]
    wk = wk_ref[...]
    wv = wv_ref[...]

    def issue(r):
        sl = slice(r * _RC, (r + 1) * _RC)
        xr = x_ref[sl, :]
        q4 = _dot(xr, wq)
        k4 = _dot(xr, wk)
        v4 = _dot(xr, wv)
        kc4 = _dot(Pc, k4, precision=_HIGH)
        vc4 = _dot(Pc, v4, precision=_HIGH)
        return q4, k4, v4, kc4, vc4

    def flush(r, q4, k4, v4, kc4, vc4):
        sl = slice(r * _RC, (r + 1) * _RC)
        bl = slice(r * nb, (r + 1) * nb)
        ones = jnp.where(
            jax.lax.broadcasted_iota(jnp.int32, (_RC, _DH), 1) == 0,
            1.0, 0.0).astype(jnp.bfloat16)
        for j in range(4):
            cl = slice(j * _DH, (j + 1) * _DH)
            hj = 4 * grp + j
            q_scr[hj, sl, :] = _b16(q4[:, cl] * _SCALE)
            kb_scr[hj, sl, :] = _b16(k4[:, cl])
            vb_scr[hj, sl, 0:_DH] = _b16(v4[:, cl])
            vb_scr[hj, sl, _DH:] = ones
            kc_scr[hj, bl, :] = kc4[:, cl]
            vc_scr[hj, bl, :] = vc4[:, cl]

        @pl.when(grp == 0)
        def _gates():
            g_scr[sl, :] = jax.nn.sigmoid(_dot(x_ref[sl, :], wg_ref[...]))

    pend = issue(0)
    for r in range(_S // _RC):
        nxt = issue(r + 1) if r + 1 < _S // _RC else None
        flush(r, *pend)
        pend = nxt


_NG = _H // 4


def _attn_body(x_ref, wq_ref, wk_ref, wv_ref, wg_ref, wo_ref,
               out_ref, q_scr, kb_scr, vb_scr, kc_scr, vc_scr, g_scr,
               qa_scr, ka_scr, oh_scr, cat_scr):
    i = pl.program_id(0)

    @pl.when(i < _NG)
    def _proj():
        _proj_group(i, x_ref, wq_ref, wk_ref, wv_ref, wg_ref,
                    q_scr, kb_scr, vb_scr, kc_scr, vc_scr, g_scr)

    @pl.when(jnp.logical_and(i >= _NG, i < _NG + _H))
    def _attend():
        _attn_head(i - _NG, q_scr, kb_scr, vb_scr, kc_scr, vc_scr, g_scr,
                   qa_scr, ka_scr, oh_scr)

    @pl.when(i >= _NG + _H)
    def _project():
        c = i - _NG - _H
        r0 = pl.multiple_of(c * _CH, _CH)
        for j in range(_H):
            cat_scr[:, j * _DH:(j + 1) * _DH] = oh_scr[j, pl.ds(r0, _CH), :]
        out_ref[...] = _dot(cat_scr[...], wo_ref[...])


def _attn_head(h, q_scr, kb_scr, vb_scr, kc_scr, vc_scr, g_scr,
               qa_scr, ka_scr, oh_scr):

    pos = jax.lax.broadcasted_iota(jnp.int32, (_S, 1), 0)
    jb = jax.lax.broadcasted_iota(jnp.int32, (1, _NBC), 1)

    kc = kc_scr[h]
    vc = vc_scr[h]

    simc = _dotT(q_scr[h], _b16(kc))
    maskc = (_CBS * jb + (_CBS - 1)) <= pos
    simcm = jnp.where(maskc, simc, _NEG)

    p_row = jax.lax.broadcasted_iota(jnp.int32, (_NBC, _S), 0)
    p_col = jax.lax.broadcasted_iota(jnp.int32, (_NBC, _S), 1)
    P = jnp.where(p_col // _CBS == p_row, 1.0 / _CBS, 0.0)
    score_t = jax.lax.dot_general(
        simcm, P, (((0,), (1,)), ((), ())),
        precision=_HIGH, preferred_element_type=jnp.float32)
    jb_t = jax.lax.broadcasted_iota(jnp.int32, (_NBC, _NQ), 0)
    jq_t = jax.lax.broadcasted_iota(jnp.int32, (_NBC, _NQ), 1)
    score_t = jnp.where(jb_t <= jq_t, score_t, _NEG)
    score_t = jnp.where(jb_t == jq_t, 1e9, score_t)
    work = score_t
    for _ in range(_NSEL - 1):
        m = jnp.max(work, axis=0, keepdims=True)
        work = jnp.where(work >= m, -3e9, work)
    thresh_t = jnp.max(work, axis=0, keepdims=True)
    selmask_t = jnp.logical_and(score_t >= thresh_t, jb_t <= jq_t)
    sel_t = selmask_t.astype(jnp.float32)
    sel_f = jnp.transpose(sel_t)
    a_rows = jnp.broadcast_to(sel_f.reshape(_NQ, 1, _NBC),
                              (_NQ, _SBS, _NBC)).reshape(_S, _NBC)
    ek_row = jax.lax.broadcasted_iota(jnp.int32, (_S, _NBC), 0)
    ek_col = jax.lax.broadcasted_iota(jnp.int32, (_S, _NBC), 1)
    ek = (ek_row // _SBS == ek_col).astype(jnp.float32)
    qa_scr[:, 0:_DH] = q_scr[h]
    qa_scr[:, _DH:] = _b16((a_rows - 1.0) * 1e9)
    ka_scr[:, 0:_DH] = kb_scr[h]
    ka_scr[:, _DH:] = _b16(ek)

    onesc = jnp.where(
        jax.lax.broadcasted_iota(jnp.int32, (_NBC, _DH), 1) == 0,
        1.0, 0.0).astype(jnp.bfloat16)
    vce = jnp.concatenate([_b16(vc), onesc], axis=1)
    ec = jnp.exp(simcm)
    avc = _dot(_b16(ec), vce)
    outc = avc[:, 0:_DH] / avc[:, _DH:_DH + 1]
    outc = jnp.where(pos >= (_CBS - 1), outc, 0.0)

    g = g_scr[...]
    gl = jax.lax.broadcasted_iota(jnp.int32, (1, 3 * _H), 1)
    g0 = jnp.sum(jnp.where(gl == 3 * h, g, 0.0), axis=1, keepdims=True)
    g1 = jnp.sum(jnp.where(gl == 3 * h + 1, g, 0.0), axis=1, keepdims=True)
    g2 = jnp.sum(jnp.where(gl == 3 * h + 2, g, 0.0), axis=1, keepdims=True)

    def issue(c):
        sl = slice(c * _CH, (c + 1) * _CH)
        kk = (c + 1) * _CH
        w0 = max(0, c * _CH - _W)
        sims = _dotT(qa_scr[sl, :], ka_scr[0:kk, :])
        simw = _dotT(q_scr[h, sl, :], kb_scr[h, w0:kk, :])
        return sims, simw

    def process(c, sims, simw):
        sl = slice(c * _CH, (c + 1) * _CH)
        kk = (c + 1) * _CH
        w0 = max(0, c * _CH - _W)
        d0 = c * _CH
        qpos = d0 + jax.lax.broadcasted_iota(jnp.int32, (_CH, 1), 0)
        dpos = d0 + jax.lax.broadcasted_iota(jnp.int32, (_CH, _CH), 1)
        ed = jnp.exp(jnp.where(dpos <= qpos, sims[:, d0:kk], _NEG))
        avs = _dot(_b16(ed), vb_scr[h, d0:kk, :])
        if c > 0:
            el = jnp.exp(sims[:, 0:d0])
            avs = avs + _dot(_b16(el), vb_scr[h, 0:d0, :])
        kposw = w0 + jax.lax.broadcasted_iota(jnp.int32, (_CH, kk - w0), 1)
        bandw = jnp.logical_and(kposw <= qpos, kposw > qpos - _W)
        ew = jnp.exp(jnp.where(bandw, simw, _NEG))
        avw = _dot(_b16(ew), vb_scr[h, w0:kk, :])
        oh_scr[h, sl, :] = _b16(
            g0[sl, :] * outc[sl, :]
            + (g1[sl, :] / avs[:, _DH:_DH + 1]) * avs[:, 0:_DH]
            + (g2[sl, :] / avw[:, _DH:_DH + 1]) * avw[:, 0:_DH])

    pend = issue(0)
    for c in range(_NC):
        nxt = issue(c + 1) if c + 1 < _NC else None
        process(c, *pend)
        pend = nxt


def _run(x2, wq2, wk2, wv2, Wg, Wo, interpret=False):
    _gm = _NG - 1
    out = pl.pallas_call(
        _attn_body,
        grid=(_NG + _H + _NC,),
        in_specs=[
            pl.BlockSpec((_S, _DIM), lambda i: (0, 0)),
            pl.BlockSpec((_DIM, 4 * _DH), lambda i: (0, jnp.minimum(i, _gm))),
            pl.BlockSpec((_DIM, 4 * _DH), lambda i: (0, jnp.minimum(i, _gm))),
            pl.BlockSpec((_DIM, 4 * _DH), lambda i: (0, jnp.minimum(i, _gm))),
            pl.BlockSpec((_DIM, 3 * _H), lambda i: (0, 0)),
            pl.BlockSpec((_H * _DH, _DIM), lambda i: (0, 0)),
        ],
        out_specs=pl.BlockSpec((_CH, _DIM),
                               lambda i: (jnp.maximum(i - _NG - _H, 0), 0)),
        out_shape=jax.ShapeDtypeStruct((_S, _DIM), jnp.float32),
        scratch_shapes=[
            pltpu.VMEM((_H, _S, _DH), jnp.bfloat16),
            pltpu.VMEM((_H, _S, _DH), jnp.bfloat16),
            pltpu.VMEM((_H, _S, 2 * _DH), jnp.bfloat16),
            pltpu.VMEM((_H, _NBC, _DH), jnp.float32),
            pltpu.VMEM((_H, _NBC, _DH), jnp.float32),
            pltpu.VMEM((_S, 3 * _H), jnp.float32),
            pltpu.VMEM((_S, _DH + _NBC), jnp.bfloat16),
            pltpu.VMEM((_S, _DH + _NBC), jnp.bfloat16),
            pltpu.VMEM((_H, _S, _DH), jnp.bfloat16),
            pltpu.VMEM((_CH, _H * _DH), jnp.bfloat16),
        ],
        interpret=interpret,
    )(x2, wq2, wk2, wv2, Wg, Wo)
    return out


def kernel(x, Wq, Wk, Wv, Wg, Wo):
    B, S, _ = x.shape
    out = _run(_b16(x.reshape(S, _DIM)), _b16(Wq), _b16(Wk), _b16(Wv),
               _b16(Wg), _b16(Wo))
    return out.reshape(B, S, _DIM)


# speedup vs baseline: 1.0318x; 1.0318x over previous
import jax
import jax.numpy as jnp
from jax.experimental import pallas as pl
from jax.experimental.pallas import tpu as pltpu

_DIM = 1024
_H = 16
_DH = 64
_W = 64
_CBS = 32
_SBS = 32
_NSEL = 16
_S = 2048
_NBC = _S // _CBS
_NQ = _S // _SBS
_CH = 256
_NC = _S // _CH
_SCALE = _DH ** -0.5
_NEG = -1e9

_HIGH = jax.lax.Precision.HIGHEST
_DEF = jax.lax.Precision.DEFAULT


def _dotT(a, b, precision=_DEF):
    return jax.lax.dot_general(a, b, (((1,), (1,)), ((), ())),
                               precision=precision,
                               preferred_element_type=jnp.float32)


def _dot(a, b, precision=_DEF):
    return jax.lax.dot_general(a, b, (((1,), (0,)), ((), ())),
                               precision=precision,
                               preferred_element_type=jnp.float32)


def _b16(a):
    return a.astype(jnp.bfloat16)


def _proj_body(x_ref, wq_ref, wk_ref, wv_ref, wg_ref,
               q_ref, kb_ref, vb_ref, kc_ref, vc_ref, g_ref, xb_scr):
    grp = pl.program_id(0)
    _RC = 512
    nb = _RC // _CBS
    pr = jax.lax.broadcasted_iota(jnp.int32, (nb, _RC), 0)
    pc = jax.lax.broadcasted_iota(jnp.int32, (nb, _RC), 1)
    Pc = jnp.where(pc // _CBS == pr, 1.0 / _CBS, 0.0)

    @pl.when(grp == 0)
    def _castx():
        for r in range(_S // _RC):
            sl = slice(r * _RC, (r + 1) * _RC)
            xb_scr[sl, :] = _b16(x_ref[sl, :])

    wq = _b16(wq_ref[...])
    wk = _b16(wk_ref[...])
    wv = _b16(wv_ref[...])

    def issue(r):
        sl = slice(r * _RC, (r + 1) * _RC)
        xr = xb_scr[sl, :]
        q4 = _dot(xr, wq)
        k4 = _dot(xr, wk)
        v4 = _dot(xr, wv)
        kc4 = _dot(Pc, k4, precision=_HIGH)
        vc4 = _dot(Pc, v4, precision=_HIGH)
        return q4, k4, v4, kc4, vc4

    def flush(r, q4, k4, v4, kc4, vc4):
        sl = slice(r * _RC, (r + 1) * _RC)
        bl = slice(r * nb, (r + 1) * nb)
        ones = jnp.where(
            jax.lax.broadcasted_iota(jnp.int32, (_RC, _DH), 1) == 0,
            1.0, 0.0).astype(jnp.bfloat16)
        for j in range(4):
            cl = slice(j * _DH, (j + 1) * _DH)
            q_ref[j, sl, :] = _b16(q4[:, cl] * _SCALE)
            kb_ref[j, sl, :] = _b16(k4[:, cl])
            vb_ref[j, sl, 0:_DH] = _b16(v4[:, cl])
            vb_ref[j, sl, _DH:] = ones
            kc_ref[j, bl, :] = kc4[:, cl]
            vc_ref[j, bl, :] = vc4[:, cl]

        @pl.when(grp == 0)
        def _gates():
            g_ref[sl, :] = jax.nn.sigmoid(
                _dot(xb_scr[sl, :], _b16(wg_ref[...])))

    pend = issue(0)
    for r in range(_S // _RC):
        nxt = issue(r + 1) if r + 1 < _S // _RC else None
        flush(r, *pend)
        pend = nxt


def _attn_body(q_ref, kb_ref, vb_ref, kc_ref, vc_ref, g_ref, wo_ref,
               out_ref, qa_scr, ka_scr, oh_scr, cat_scr, wob_scr):
    i = pl.program_id(0)

    @pl.when(i == 0)
    def _castwo():
        wob_scr[...] = _b16(wo_ref[...])

    @pl.when(i >= _H)
    def _project():
        c = i - _H
        r0 = pl.multiple_of(c * _CH, _CH)
        for j in range(_H):
            cat_scr[:, j * _DH:(j + 1) * _DH] = oh_scr[j, pl.ds(r0, _CH), :]
        out_ref[...] = _dot(cat_scr[...], wob_scr[...])

    @pl.when(i < _H)
    def _attend():
        _attn_head(i, q_ref, kb_ref, vb_ref, kc_ref, vc_ref, g_ref,
                   qa_scr, ka_scr, oh_scr)


def _attn_head(h, q_ref, kb_ref, vb_ref, kc_ref, vc_ref, g_ref,
               qa_scr, ka_scr, oh_scr):

    pos = jax.lax.broadcasted_iota(jnp.int32, (_S, 1), 0)
    jb = jax.lax.broadcasted_iota(jnp.int32, (1, _NBC), 1)

    kc = kc_ref[0]
    vc = vc_ref[0]

    simc = _dotT(q_ref[0], _b16(kc))
    maskc = (_CBS * jb + (_CBS - 1)) <= pos
    simcm = jnp.where(maskc, simc, _NEG)

    p_row = jax.lax.broadcasted_iota(jnp.int32, (_NBC, _S), 0)
    p_col = jax.lax.broadcasted_iota(jnp.int32, (_NBC, _S), 1)
    P = jnp.where(p_col // _CBS == p_row, 1.0 / _CBS, 0.0)
    score_t = jax.lax.dot_general(
        simcm, P, (((0,), (1,)), ((), ())),
        precision=_HIGH, preferred_element_type=jnp.float32)
    jb_t = jax.lax.broadcasted_iota(jnp.int32, (_NBC, _NQ), 0)
    jq_t = jax.lax.broadcasted_iota(jnp.int32, (_NBC, _NQ), 1)
    score_t = jnp.where(jb_t <= jq_t, score_t, _NEG)
    score_t = jnp.where(jb_t == jq_t, 1e9, score_t)
    work = score_t
    for _ in range(_NSEL - 1):
        m = jnp.max(work, axis=0, keepdims=True)
        work = jnp.where(work >= m, -3e9, work)
    thresh_t = jnp.max(work, axis=0, keepdims=True)
    selmask_t = jnp.logical_and(score_t >= thresh_t, jb_t <= jq_t)
    sel_t = selmask_t.astype(jnp.float32)
    sel_f = jnp.transpose(sel_t)
    a_rows = jnp.broadcast_to(sel_f.reshape(_NQ, 1, _NBC),
                              (_NQ, _SBS, _NBC)).reshape(_S, _NBC)
    ek_row = jax.lax.broadcasted_iota(jnp.int32, (_S, _NBC), 0)
    ek_col = jax.lax.broadcasted_iota(jnp.int32, (_S, _NBC), 1)
    ek = (ek_row // _SBS == ek_col).astype(jnp.float32)
    qa_scr[:, 0:_DH] = q_ref[0]
    qa_scr[:, _DH:] = _b16((a_rows - 1.0) * 1e9)
    ka_scr[:, 0:_DH] = kb_ref[0]
    ka_scr[:, _DH:] = _b16(ek)

    onesc = jnp.where(
        jax.lax.broadcasted_iota(jnp.int32, (_NBC, _DH), 1) == 0,
        1.0, 0.0).astype(jnp.bfloat16)
    vce = jnp.concatenate([_b16(vc), onesc], axis=1)
    ec = jnp.exp(simcm)
    avc = _dot(_b16(ec), vce)
    outc = avc[:, 0:_DH] / avc[:, _DH:_DH + 1]
    outc = jnp.where(pos >= (_CBS - 1), outc, 0.0)

    g = g_ref[...]
    gl = jax.lax.broadcasted_iota(jnp.int32, (1, 3 * _H), 1)
    g0 = jnp.sum(jnp.where(gl == 3 * h, g, 0.0), axis=1, keepdims=True)
    g1 = jnp.sum(jnp.where(gl == 3 * h + 1, g, 0.0), axis=1, keepdims=True)
    g2 = jnp.sum(jnp.where(gl == 3 * h + 2, g, 0.0), axis=1, keepdims=True)

    def issue(c):
        sl = slice(c * _CH, (c + 1) * _CH)
        kk = (c + 1) * _CH
        w0 = max(0, c * _CH - _W)
        sims = _dotT(qa_scr[sl, :], ka_scr[0:kk, :])
        simw = _dotT(q_ref[0, sl, :], kb_ref[0, w0:kk, :])
        return sims, simw

    def process(c, sims, simw):
        sl = slice(c * _CH, (c + 1) * _CH)
        kk = (c + 1) * _CH
        w0 = max(0, c * _CH - _W)
        d0 = c * _CH
        qpos = d0 + jax.lax.broadcasted_iota(jnp.int32, (_CH, 1), 0)
        dpos = d0 + jax.lax.broadcasted_iota(jnp.int32, (_CH, _CH), 1)
        ed = jnp.exp(jnp.where(dpos <= qpos, sims[:, d0:kk], _NEG))
        avs = _dot(_b16(ed), vb_ref[0, d0:kk, :])
        if c > 0:
            el = jnp.exp(sims[:, 0:d0])
            avs = avs + _dot(_b16(el), vb_ref[0, 0:d0, :])
        kposw = w0 + jax.lax.broadcasted_iota(jnp.int32, (_CH, kk - w0), 1)
        bandw = jnp.logical_and(kposw <= qpos, kposw > qpos - _W)
        ew = jnp.exp(jnp.where(bandw, simw, _NEG))
        avw = _dot(_b16(ew), vb_ref[0, w0:kk, :])
        oh_scr[h, sl, :] = _b16(
            g0[sl, :] * outc[sl, :]
            + (g1[sl, :] / avs[:, _DH:_DH + 1]) * avs[:, 0:_DH]
            + (g2[sl, :] / avw[:, _DH:_DH + 1]) * avw[:, 0:_DH])

    pend = issue(0)
    for c in range(_NC):
        nxt = issue(c + 1) if c + 1 < _NC else None
        process(c, *pend)
        pend = nxt


def _run(x2, wq2, wk2, wv2, Wg, Wo):
    q3, k3b, v3b, kc3, vc3, g2 = pl.pallas_call(
        _proj_body,
        grid=(_H // 4,),
        in_specs=[
            pl.BlockSpec((_S, _DIM), lambda g: (0, 0)),
            pl.BlockSpec((_DIM, 4 * _DH), lambda g: (0, g)),
            pl.BlockSpec((_DIM, 4 * _DH), lambda g: (0, g)),
            pl.BlockSpec((_DIM, 4 * _DH), lambda g: (0, g)),
            pl.BlockSpec((_DIM, 3 * _H), lambda g: (0, 0)),
        ],
        out_specs=[
            pl.BlockSpec((4, _S, _DH), lambda g: (g, 0, 0)),
            pl.BlockSpec((4, _S, _DH), lambda g: (g, 0, 0)),
            pl.BlockSpec((4, _S, 2 * _DH), lambda g: (g, 0, 0)),
            pl.BlockSpec((4, _NBC, _DH), lambda g: (g, 0, 0)),
            pl.BlockSpec((4, _NBC, _DH), lambda g: (g, 0, 0)),
            pl.BlockSpec((_S, 3 * _H), lambda g: (0, 0)),
        ],
        out_shape=[
            jax.ShapeDtypeStruct((_H, _S, _DH), jnp.bfloat16),
            jax.ShapeDtypeStruct((_H, _S, _DH), jnp.bfloat16),
            jax.ShapeDtypeStruct((_H, _S, 2 * _DH), jnp.bfloat16),
            jax.ShapeDtypeStruct((_H, _NBC, _DH), jnp.float32),
            jax.ShapeDtypeStruct((_H, _NBC, _DH), jnp.float32),
            jax.ShapeDtypeStruct((_S, 3 * _H), jnp.float32),
        ],
        scratch_shapes=[pltpu.VMEM((_S, _DIM), jnp.bfloat16)],
    )(x2, wq2, wk2, wv2, Wg)

    _hm = _H - 1
    out = pl.pallas_call(
        _attn_body,
        grid=(_H + _NC,),
        in_specs=[
            pl.BlockSpec((1, _S, _DH), lambda i: (jnp.minimum(i, _hm), 0, 0)),
            pl.BlockSpec((1, _S, _DH), lambda i: (jnp.minimum(i, _hm), 0, 0)),
            pl.BlockSpec((1, _S, 2 * _DH), lambda i: (jnp.minimum(i, _hm), 0, 0)),
            pl.BlockSpec((1, _NBC, _DH), lambda i: (jnp.minimum(i, _hm), 0, 0)),
            pl.BlockSpec((1, _NBC, _DH), lambda i: (jnp.minimum(i, _hm), 0, 0)),
            pl.BlockSpec((_S, 3 * _H), lambda i: (0, 0)),
            pl.BlockSpec((_H * _DH, _DIM), lambda i: (0, 0)),
        ],
        out_specs=pl.BlockSpec((_CH, _DIM),
                               lambda i: (jnp.maximum(i - _H, 0), 0)),
        out_shape=jax.ShapeDtypeStruct((_S, _DIM), jnp.float32),
        scratch_shapes=[
            pltpu.VMEM((_S, _DH + _NBC), jnp.bfloat16),
            pltpu.VMEM((_S, _DH + _NBC), jnp.bfloat16),
            pltpu.VMEM((_H, _S, _DH), jnp.bfloat16),
            pltpu.VMEM((_CH, _H * _DH), jnp.bfloat16),
            pltpu.VMEM((_H * _DH, _DIM), jnp.bfloat16),
        ],
    )(q3, k3b, v3b, kc3, vc3, g2, Wo)
    return out


def kernel(x, Wq, Wk, Wv, Wg, Wo):
    B, S, _ = x.shape
    out = _run(x.reshape(S, _DIM), Wq, Wk, Wv, Wg, Wo)
    return out.reshape(B, S, _DIM)
